# baseline (speedup 1.0000x reference)
import jax
import jax.numpy as jnp
from jax.experimental import pallas as pl
from jax.experimental.pallas import tpu as pltpu

_CHUNKS = (64, 128, 128, 128, 128, 128, 128, 128, 64)
_MAX = max(_CHUNKS)


def _body(x_hbm, w_ref, o_hbm, inb, outb, insem, outsem):
    offs = []
    o = 0
    for c in _CHUNKS:
        offs.append(o)
        o += c
    n = len(_CHUNKS)

    def in_copy(i):
        s = i % 2
        return pltpu.make_async_copy(
            x_hbm.at[pl.ds(offs[i], _CHUNKS[i])],
            inb.at[s, pl.ds(0, _CHUNKS[i])],
            insem.at[s],
        )

    def out_copy(i):
        s = i % 2
        return pltpu.make_async_copy(
            outb.at[s, pl.ds(0, _CHUNKS[i])],
            o_hbm.at[pl.ds(offs[i], _CHUNKS[i])],
            outsem.at[s],
        )

    in_copy(0).start()
    in_copy(1).start()
    w = w_ref[...][None, :, :]
    for i in range(n):
        s = i % 2
        in_copy(i).wait()
        if i >= 2:
            out_copy(i - 2).wait()
        outb[s, 0:_CHUNKS[i]] = inb[s, 0:_CHUNKS[i]] + w
        out_copy(i).start()
        if i + 2 < n:
            in_copy(i + 2).start()
    out_copy(n - 2).wait()
    out_copy(n - 1).wait()


def kernel(x, pos_emb_weight):
    B, S, D = x.shape
    table = pos_emb_weight[:S]
    return pl.pallas_call(
        _body,
        in_specs=[
            pl.BlockSpec(memory_space=pl.ANY),
            pl.BlockSpec((S, D), lambda: (0, 0)),
        ],
        out_specs=pl.BlockSpec(memory_space=pl.ANY),
        out_shape=jax.ShapeDtypeStruct((B, S, D), x.dtype),
        scratch_shapes=[
            pltpu.VMEM((2, _MAX, S, D), jnp.float32),
            pltpu.VMEM((2, _MAX, S, D), jnp.float32),
            pltpu.SemaphoreType.DMA((2,)),
            pltpu.SemaphoreType.DMA((2,)),
        ],
    )(x, table)
